# Initial kernel scaffold; baseline (speedup 1.0000x reference)
import jax
import jax.numpy as jnp
from jax.experimental import pallas as pl


def kernel(pos, edge_index, atom_types, type_embeddings, W_basis):
    raise NotImplementedError("write your pallas kernel here")


# speedup vs baseline: 6.6835x; 6.6835x over previous
import functools

import jax
import jax.numpy as jnp
from jax import lax
from jax.experimental import pallas as pl
from jax.experimental.pallas import tpu as pltpu
from jax.experimental.pallas import tpu_sc as plsc

N_NODES = 50000
N_EDGES = 800000
NUM_TYPES = 32
NUM_BASIS = 8
R_MAX = 5.0
OUT_W = NUM_BASIS + 2 * NUM_TYPES

NC, NS, L = 2, 16, 16
NW = NC * NS
B = 640
NCHUNK = N_EDGES // B
GROUPS = B // L
KSUB = B // 128
GEOM_W = 16
W_OFF = 2 * NUM_TYPES * NUM_TYPES
TEV_N = W_OFF + NUM_BASIS * NUM_BASIS

_TWO_PI = 6.283185307179586
_INV_TWO_PI = 1.0 / _TWO_PI
_SIN_C = (9.9998458677e-01, -1.6663258204e-01, 8.3123829338e-03,
          -1.9316182196e-04, 2.1732100681e-06)
_COS_C = (9.9999944342e-01, -4.9999558037e-01, 4.1661031574e-02,
          -1.3862743260e-03, 2.4253137751e-05, -2.2193694177e-07)


def _rsqrt(l2):
    i = plsc.bitcast(l2, jnp.int32)
    y = plsc.bitcast(jnp.int32(0x5F3759DF) - (i >> 1), jnp.float32)
    for _ in range(3):
        y = y * (1.5 - 0.5 * l2 * y * y)
    return y


def _sc_kernel(geom_hbm, src_hbm, dst_hbm, te_hbm, out_hbm,
               gsrc, gdst, tev, obuf, sem, *idx_bufs):
    isrc = idx_bufs[:KSUB]
    idst = idx_bufs[KSUB:]
    wid = lax.axis_index("s") * NC + lax.axis_index("c")
    pltpu.sync_copy(te_hbm, tev)
    iota = lax.iota(jnp.int32, L)

    def full_i(v):
        return jnp.full((L,), v, jnp.int32)

    @pl.loop(wid, NCHUNK, step=NW)
    def _chunk(c):
        base = c * B
        for k in range(KSUB):
            pltpu.sync_copy(src_hbm.at[pl.ds(base + k * 128, 128)], isrc[k])
            pltpu.sync_copy(dst_hbm.at[pl.ds(base + k * 128, 128)], idst[k])
        for k in range(KSUB):
            pltpu.async_copy(geom_hbm.at[isrc[k]],
                             gsrc.at[pl.ds(k * 128, 128)],
                             sem).wait()
            pltpu.async_copy(geom_hbm.at[idst[k]],
                             gdst.at[pl.ds(k * 128, 128)],
                             sem).wait()

        @pl.loop(0, GROUPS)
        def _group(g):
            rows = g * L + iota
            xs = plsc.load_gather(gsrc, [rows, full_i(0)])
            ys = plsc.load_gather(gsrc, [rows, full_i(1)])
            zs = plsc.load_gather(gsrc, [rows, full_i(2)])
            ats = plsc.load_gather(gsrc, [rows, full_i(3)]).astype(jnp.int32)
            xd = plsc.load_gather(gdst, [rows, full_i(0)])
            yd = plsc.load_gather(gdst, [rows, full_i(1)])
            zd = plsc.load_gather(gdst, [rows, full_i(2)])
            atd = plsc.load_gather(gdst, [rows, full_i(3)]).astype(jnp.int32)
            dx = xd - xs
            dy = yd - ys
            dz = zd - zs
            l2 = dx * dx + dy * dy + dz * dz + 1e-12
            inv = _rsqrt(l2)
            x = l2 * inv
            theta = x * (jnp.pi / R_MAX)
            q = (theta * _INV_TWO_PI + 0.5).astype(jnp.int32).astype(jnp.float32)
            th = theta - q * _TWO_PI
            t2 = th * th
            s1 = th * (_SIN_C[0] + t2 * (_SIN_C[1] + t2 * (_SIN_C[2]
                       + t2 * (_SIN_C[3] + t2 * _SIN_C[4]))))
            c1 = (_COS_C[0] + t2 * (_COS_C[1] + t2 * (_COS_C[2]
                  + t2 * (_COS_C[3] + t2 * (_COS_C[4] + t2 * _COS_C[5])))))
            scale = (2.0 / R_MAX) * inv
            two_c1 = 2.0 * c1
            sins = [s1]
            for _ in range(NUM_BASIS - 1):
                prev = sins[-2] if len(sins) >= 2 else jnp.zeros_like(s1)
                sins.append(two_c1 * sins[-1] - prev)
            bas = [s * scale for s in sins]
            orow = g * (L * OUT_W) + iota * OUT_W
            for j in range(NUM_BASIS):
                acc = bas[0] * plsc.load_gather(tev, [full_i(W_OFF + j)])
                for n in range(1, NUM_BASIS):
                    acc = acc + bas[n] * plsc.load_gather(
                        tev, [full_i(W_OFF + n * NUM_BASIS + j)])
                plsc.store_scatter(obuf, [orow + j], acc)
            base_s = ats * NUM_TYPES
            base_d = atd * NUM_TYPES + NUM_TYPES * NUM_TYPES
            for j in range(NUM_TYPES):
                v = plsc.load_gather(tev, [base_s + j])
                plsc.store_scatter(obuf, [orow + (NUM_BASIS + j)], v)
                v2 = plsc.load_gather(tev, [base_d + j])
                plsc.store_scatter(obuf, [orow + (NUM_BASIS + NUM_TYPES + j)],
                                   v2)

        pltpu.sync_copy(obuf, out_hbm.at[pl.ds(base * OUT_W, B * OUT_W)])


@jax.jit
def kernel(pos, edge_index, atom_types, type_embeddings, W_basis):
    at32 = atom_types.astype(jnp.int32)
    geom = jnp.concatenate(
        [pos, at32.astype(jnp.float32)[:, None],
         jnp.zeros((N_NODES, GEOM_W - 4), jnp.float32)], axis=1)
    ei = edge_index.astype(jnp.int32)
    src = ei[0]
    dst = ei[1]
    te = jnp.concatenate([type_embeddings.reshape(-1), W_basis.reshape(-1)])

    mesh = plsc.VectorSubcoreMesh(core_axis_name="c", subcore_axis_name="s",
                                  num_cores=NC, num_subcores=NS)
    f = pl.kernel(
        _sc_kernel,
        out_type=jax.ShapeDtypeStruct((N_EDGES * OUT_W,), jnp.float32),
        mesh=mesh,
        compiler_params=pltpu.CompilerParams(needs_layout_passes=False, use_tc_tiling_on_sc=False),
        scratch_types=[
            pltpu.VMEM((B, GEOM_W), jnp.float32),
            pltpu.VMEM((B, GEOM_W), jnp.float32),
            pltpu.VMEM((TEV_N,), jnp.float32),
            pltpu.VMEM((B * OUT_W,), jnp.float32),
            pltpu.SemaphoreType.DMA,
        ] + [pltpu.VMEM((128,), jnp.int32) for _ in range(2 * KSUB)],
    )
    return f(geom, src, dst, te).reshape(N_EDGES, OUT_W)


# speedup vs baseline: 7.6223x; 1.1405x over previous
import functools

import jax
import jax.numpy as jnp
from jax import lax
from jax.experimental import pallas as pl
from jax.experimental.pallas import tpu as pltpu
from jax.experimental.pallas import tpu_sc as plsc

N_NODES = 50000
N_EDGES = 800000
NUM_TYPES = 32
NUM_BASIS = 8
R_MAX = 5.0
OUT_W = NUM_BASIS + 2 * NUM_TYPES

NC, NS, L = 2, 16, 16
NW = NC * NS
B = 640
NCHUNK = N_EDGES // B
GROUPS = B // L
KSUB = B // 128
GEOM_W = 16
W_OFF = 2 * NUM_TYPES * NUM_TYPES
TEV_N = W_OFF + NUM_BASIS * NUM_BASIS

_TWO_PI = 6.283185307179586
_INV_TWO_PI = 1.0 / _TWO_PI
_SIN_C = (9.9998458677e-01, -1.6663258204e-01, 8.3123829338e-03,
          -1.9316182196e-04, 2.1732100681e-06)
_COS_C = (9.9999944342e-01, -4.9999558037e-01, 4.1661031574e-02,
          -1.3862743260e-03, 2.4253137751e-05, -2.2193694177e-07)


def _rsqrt(l2):
    i = plsc.bitcast(l2, jnp.int32)
    y = plsc.bitcast(jnp.int32(0x5F3759DF) - (i >> 1), jnp.float32)
    for _ in range(3):
        y = y * (1.5 - 0.5 * l2 * y * y)
    return y


def _sc_kernel(geom_hbm, src_hbm, dst_hbm, te_hbm, out_hbm,
               gsrc, gdst, tev, obuf, sem, isrc, idst):
    wid = lax.axis_index("s") * NC + lax.axis_index("c")
    pltpu.sync_copy(te_hbm, tev)
    iota = lax.iota(jnp.int32, L)

    def full_i(v):
        return jnp.full((L,), v, jnp.int32)

    @pl.loop(wid, NCHUNK, step=NW)
    def _chunk(c):
        base = c * B
        pltpu.sync_copy(src_hbm.at[pl.ds(base, B)], isrc)
        pltpu.sync_copy(dst_hbm.at[pl.ds(base, B)], idst)
        cp1 = pltpu.async_copy(geom_hbm.at[isrc], gsrc, sem)
        cp2 = pltpu.async_copy(geom_hbm.at[idst], gdst, sem)
        cp1.wait()
        cp2.wait()

        @pl.loop(0, GROUPS)
        def _group(g):
            rows = g * L + iota
            xs = plsc.load_gather(gsrc, [rows, full_i(0)])
            ys = plsc.load_gather(gsrc, [rows, full_i(1)])
            zs = plsc.load_gather(gsrc, [rows, full_i(2)])
            ats = plsc.load_gather(gsrc, [rows, full_i(3)]).astype(jnp.int32)
            xd = plsc.load_gather(gdst, [rows, full_i(0)])
            yd = plsc.load_gather(gdst, [rows, full_i(1)])
            zd = plsc.load_gather(gdst, [rows, full_i(2)])
            atd = plsc.load_gather(gdst, [rows, full_i(3)]).astype(jnp.int32)
            dx = xd - xs
            dy = yd - ys
            dz = zd - zs
            l2 = dx * dx + dy * dy + dz * dz + 1e-12
            inv = _rsqrt(l2)
            x = l2 * inv
            theta = x * (jnp.pi / R_MAX)
            q = (theta * _INV_TWO_PI + 0.5).astype(jnp.int32).astype(jnp.float32)
            th = theta - q * _TWO_PI
            t2 = th * th
            s1 = th * (_SIN_C[0] + t2 * (_SIN_C[1] + t2 * (_SIN_C[2]
                       + t2 * (_SIN_C[3] + t2 * _SIN_C[4]))))
            c1 = (_COS_C[0] + t2 * (_COS_C[1] + t2 * (_COS_C[2]
                  + t2 * (_COS_C[3] + t2 * (_COS_C[4] + t2 * _COS_C[5])))))
            scale = (2.0 / R_MAX) * inv
            two_c1 = 2.0 * c1
            sins = [s1]
            for _ in range(NUM_BASIS - 1):
                prev = sins[-2] if len(sins) >= 2 else jnp.zeros_like(s1)
                sins.append(two_c1 * sins[-1] - prev)
            bas = [s * scale for s in sins]
            orow = g * (L * OUT_W) + iota * OUT_W
            for j in range(NUM_BASIS):
                acc = bas[0] * plsc.load_gather(tev, [full_i(W_OFF + j)])
                for n in range(1, NUM_BASIS):
                    acc = acc + bas[n] * plsc.load_gather(
                        tev, [full_i(W_OFF + n * NUM_BASIS + j)])
                plsc.store_scatter(obuf, [orow + j], acc)
            base_s = ats * NUM_TYPES
            base_d = atd * NUM_TYPES + NUM_TYPES * NUM_TYPES
            for j in range(NUM_TYPES):
                v = plsc.load_gather(tev, [base_s + j])
                plsc.store_scatter(obuf, [orow + (NUM_BASIS + j)], v)
                v2 = plsc.load_gather(tev, [base_d + j])
                plsc.store_scatter(obuf, [orow + (NUM_BASIS + NUM_TYPES + j)],
                                   v2)

        pltpu.sync_copy(obuf, out_hbm.at[pl.ds(base * OUT_W, B * OUT_W)])


@jax.jit
def kernel(pos, edge_index, atom_types, type_embeddings, W_basis):
    at32 = atom_types.astype(jnp.int32)
    geom = jnp.concatenate(
        [pos, at32.astype(jnp.float32)[:, None],
         jnp.zeros((N_NODES, GEOM_W - 4), jnp.float32)], axis=1)
    ei = edge_index.astype(jnp.int32)
    src = ei[0]
    dst = ei[1]
    te = jnp.concatenate([type_embeddings.reshape(-1), W_basis.reshape(-1)])

    mesh = plsc.VectorSubcoreMesh(core_axis_name="c", subcore_axis_name="s",
                                  num_cores=NC, num_subcores=NS)
    f = pl.kernel(
        _sc_kernel,
        out_type=jax.ShapeDtypeStruct((N_EDGES * OUT_W,), jnp.float32),
        mesh=mesh,
        compiler_params=pltpu.CompilerParams(needs_layout_passes=False, use_tc_tiling_on_sc=False),
        scratch_types=[
            pltpu.VMEM((B, GEOM_W), jnp.float32),
            pltpu.VMEM((B, GEOM_W), jnp.float32),
            pltpu.VMEM((TEV_N,), jnp.float32),
            pltpu.VMEM((B * OUT_W,), jnp.float32),
            pltpu.SemaphoreType.DMA,
            pltpu.VMEM((B,), jnp.int32),
            pltpu.VMEM((B,), jnp.int32),
        ],
    )
    return f(geom, src, dst, te).reshape(N_EDGES, OUT_W)
